# baseline (speedup 1.0000x reference)
import functools

import jax
import jax.numpy as jnp
from jax import lax
from jax.experimental import pallas as pl
from jax.experimental.pallas import tpu as pltpu
from jax.experimental.pallas import tpu_sc as plsc

_T = 3
_LANES = 128
_BB = 1024
_NC, _NS, _L = 2, 16, 16
_NW = _NC * _NS


def _tc_mlp_block(thr_ref, x_ref, w1_ref, b1_ref, w2_ref, b2_ref,
                  w_ref, g_ref):
    x = x_ref[...]
    h = lax.dot_general(x, w1_ref[...], (((1,), (0,)), ((), ())),
                        preferred_element_type=jnp.float32)
    h = jnp.maximum(h + b1_ref[...], 0.0)
    logits = lax.dot_general(h, w2_ref[...], (((1,), (0,)), ((), ())),
                             preferred_element_type=jnp.float32)
    logits = logits + b2_ref[...]
    m = jnp.max(logits, axis=-1, keepdims=True)
    ex = jnp.exp(logits - m)
    s = jnp.sum(ex, axis=-1, keepdims=True)
    p = ex / s
    j = lax.broadcasted_iota(jnp.int32, p.shape, 1)
    valid = j < _T
    maxp = jnp.max(p, axis=-1, keepdims=True)
    minp = jnp.min(jnp.where(valid, p, jnp.inf), axis=-1, keepdims=True)
    a = jnp.min(jnp.where(p == maxp, j, _LANES), axis=-1, keepdims=True)
    l = jnp.max(jnp.where(valid & (p == minp), j, -1), axis=-1, keepdims=True)
    use_topk = maxp < thr_ref[0]
    w_topk = jnp.where(j == l, 0.0, p)
    w_arg = jnp.where(j == a, 1.0, 0.0)
    w_ref[...] = jnp.where(use_topk, w_topk, w_arg)[:, :_L]
    g_ref[...] = p[:, :_T]


def _tc_mlp(h_t, W1, b1, W2, b2, threshold):
    B, d_model = h_t.shape
    hidden = W1.shape[1]
    w2p = jnp.zeros((hidden, _LANES), jnp.float32).at[:, :_T].set(W2)
    b2p = jnp.full((1, _LANES), -1e30, jnp.float32).at[0, :_T].set(b2)
    thr = jnp.reshape(jnp.asarray(threshold, jnp.float32), (1,))
    return pl.pallas_call(
        _tc_mlp_block,
        grid=(B // _BB,),
        in_specs=[
            pl.BlockSpec(memory_space=pltpu.SMEM),
            pl.BlockSpec((_BB, d_model), lambda i: (i, 0)),
            pl.BlockSpec((d_model, hidden), lambda i: (0, 0)),
            pl.BlockSpec((1, hidden), lambda i: (0, 0)),
            pl.BlockSpec((hidden, _LANES), lambda i: (0, 0)),
            pl.BlockSpec((1, _LANES), lambda i: (0, 0)),
        ],
        out_specs=[
            pl.BlockSpec((_BB, _L), lambda i: (i, 0)),
            pl.BlockSpec((_BB, _T), lambda i: (i, 0)),
        ],
        out_shape=[
            jax.ShapeDtypeStruct((B, _L), jnp.float32),
            jax.ShapeDtypeStruct((B, _T), jnp.float32),
        ],
        compiler_params=pltpu.CompilerParams(
            dimension_semantics=("arbitrary",),
        ),
    )(thr, h_t, W1, jnp.reshape(b1, (1, hidden)), w2p, b2p)


def _sc_combine(rows, d_model):
    tw = rows // _NW
    G = 8
    ngrp = tw // G
    ndch = d_model // _L
    mesh = plsc.VectorSubcoreMesh(core_axis_name="c", subcore_axis_name="s")

    @functools.partial(
        pl.kernel, mesh=mesh,
        out_type=jax.ShapeDtypeStruct((rows, d_model), jnp.float32),
        scratch_types=[
            pltpu.VMEM((_T, d_model), jnp.float32),
            pltpu.VMEM((tw, _L), jnp.float32),
            pltpu.VMEM((2, G, d_model), jnp.float32),
            pltpu.SemaphoreType.DMA((2,)),
        ],
    )
    def k(w_hbm, cb_hbm, e_hbm, cb_v, w_v, e_v, sem):
        wid = lax.axis_index("s") * _NC + lax.axis_index("c")
        base = wid * tw
        pltpu.sync_copy(cb_hbm, cb_v)
        pltpu.sync_copy(w_hbm.at[pl.ds(base, tw)], w_v)

        for g in range(ngrp):
            slot = g % 2
            if g >= 2:
                pltpu.make_async_copy(
                    e_v.at[slot],
                    e_hbm.at[pl.ds(base + (g - 2) * G, G)],
                    sem.at[slot],
                ).wait()

            wscal = []
            for t in range(G):
                wv = w_v[g * G + t, pl.ds(0, _L)]
                wscal.append((wv[0], wv[1], wv[2]))

            def dbody(c, _, slot=slot, wscal=wscal):
                sl = pl.ds(c * _L, _L)
                cb0 = cb_v[0, sl]
                cb1 = cb_v[1, sl]
                cb2 = cb_v[2, sl]
                for t in range(G):
                    w0, w1, w2 = wscal[t]
                    e_v[slot, t, sl] = cb0 * w0 + cb1 * w1 + cb2 * w2
                return 0

            lax.fori_loop(0, ndch, dbody, 0)
            pltpu.make_async_copy(
                e_v.at[slot],
                e_hbm.at[pl.ds(base + g * G, G)],
                sem.at[slot],
            ).start()

        for g in (ngrp - 2, ngrp - 1):
            pltpu.make_async_copy(
                e_v.at[g % 2],
                e_hbm.at[pl.ds(base + g * G, G)],
                sem.at[g % 2],
            ).wait()

    return k


@jax.jit
def kernel(h_t, W1, b1, W2, b2, codebook, threshold=0.7):
    B, d_model = h_t.shape
    w, g = _tc_mlp(h_t, W1, b1, W2, b2, threshold)
    e = _sc_combine(B, d_model)(w, codebook)
    return (e, g)


# speedup vs baseline: 1.6904x; 1.6904x over previous
import jax
import jax.numpy as jnp
from jax.experimental import pallas as pl
from jax.experimental.pallas import tpu as pltpu

_T = 3
_LANES = 128
_BB = 1024


def _router_block(thr_ref, x_ref, w1_ref, b1_ref, w2_ref, b2_ref, cb_ref,
                  e_ref, g_ref):
    x = x_ref[...]
    h = jax.lax.dot_general(x, w1_ref[...], (((1,), (0,)), ((), ())),
                            preferred_element_type=jnp.float32)
    h = jnp.maximum(h + b1_ref[...], 0.0)
    logits = jax.lax.dot_general(h, w2_ref[...], (((1,), (0,)), ((), ())),
                                 preferred_element_type=jnp.float32)
    logits = logits + b2_ref[...]

    m = jnp.max(logits, axis=-1, keepdims=True)
    ex = jnp.exp(logits - m)
    s = jnp.sum(ex, axis=-1, keepdims=True)
    p = ex / s

    j = jax.lax.broadcasted_iota(jnp.int32, p.shape, 1)
    valid = j < _T
    maxp = jnp.max(p, axis=-1, keepdims=True)
    minp = jnp.min(jnp.where(valid, p, jnp.inf), axis=-1, keepdims=True)
    a = jnp.min(jnp.where(p == maxp, j, _LANES), axis=-1, keepdims=True)
    l = jnp.max(jnp.where(valid & (p == minp), j, -1), axis=-1, keepdims=True)

    use_topk = maxp < thr_ref[0]
    w_topk = jnp.where(j == l, 0.0, p)
    w_arg = jnp.where(j == a, 1.0, 0.0)
    w = jnp.where(use_topk, w_topk, w_arg).astype(jnp.bfloat16)

    e_ref[...] = jax.lax.dot_general(w, cb_ref[...], (((1,), (0,)), ((), ())),
                                     preferred_element_type=jnp.float32)
    g_ref[...] = p[:, :_T]


@jax.jit
def kernel(h_t, W1, b1, W2, b2, codebook, threshold=0.7):
    B, d_model = h_t.shape
    hidden = W1.shape[1]
    grid = B // _BB

    w2p = jnp.zeros((hidden, _LANES), jnp.float32).at[:, :_T].set(W2)
    b2p = jnp.full((1, _LANES), -1e30, jnp.float32).at[0, :_T].set(b2)
    cbp = (jnp.zeros((_LANES, d_model), jnp.float32).at[:_T, :].set(codebook)
           .astype(jnp.bfloat16))
    thr = jnp.reshape(jnp.asarray(threshold, jnp.float32), (1,))

    e_task, g_task = pl.pallas_call(
        _router_block,
        grid=(grid,),
        in_specs=[
            pl.BlockSpec(memory_space=pltpu.SMEM),
            pl.BlockSpec((_BB, d_model), lambda i: (i, 0)),
            pl.BlockSpec((d_model, hidden), lambda i: (0, 0)),
            pl.BlockSpec((1, hidden), lambda i: (0, 0)),
            pl.BlockSpec((hidden, _LANES), lambda i: (0, 0)),
            pl.BlockSpec((1, _LANES), lambda i: (0, 0)),
            pl.BlockSpec((_LANES, d_model), lambda i: (0, 0)),
        ],
        out_specs=[
            pl.BlockSpec((_BB, d_model), lambda i: (i, 0)),
            pl.BlockSpec((_BB, _T), lambda i: (i, 0)),
        ],
        out_shape=[
            jax.ShapeDtypeStruct((B, d_model), jnp.float32),
            jax.ShapeDtypeStruct((B, _T), jnp.float32),
        ],
        compiler_params=pltpu.CompilerParams(
            dimension_semantics=("arbitrary",),
        ),
    )(thr, h_t, W1, jnp.reshape(b1, (1, hidden)), w2p, b2p, cbp)
    return (e_task, g_task)
